# baseline (speedup 1.0000x reference)
import functools

import numpy as np
import jax
import jax.numpy as jnp
from jax import lax
from jax.experimental import pallas as pl
from jax.experimental.pallas import tpu as pltpu
from jax.experimental.pallas import tpu_sc as plsc

N = 10000
E = 320000
D = 128

NC, NS, L = 2, 16, 16
NW = NC * NS
CH = 2000
CPT = 5
NPAD = 10240
RPT = NPAD // NS
NBUF = 5


def _sc_mesh():
    return plsc.VectorSubcoreMesh(core_axis_name="c", subcore_axis_name="s",
                                  num_cores=NC, num_subcores=NS)


_SC_PARAMS = pltpu.CompilerParams(use_tc_tiling_on_sc=False)


def _deg_body(ei_hbm, zeros_hbm, out_hbm, dst_v, ones_v, acc_sh):
    c = lax.axis_index("c")
    s = lax.axis_index("s")
    wid = c * NS + s
    lo = s * RPT
    pltpu.sync_copy(zeros_hbm.at[pl.ds(lo, RPT)], acc_sh.at[pl.ds(lo, RPT)])
    pltpu.sync_copy(ei_hbm.at[1, wid], dst_v)
    for i in range(CH // L):
        ones_v[pl.ds(i * L, L)] = jnp.full((L,), 1.0, jnp.float32)
    plsc.subcore_barrier()

    def chunk(j, carry):
        pltpu.sync_copy(ones_v, acc_sh.at[dst_v.at[j]], add=True)
        return carry

    lax.fori_loop(0, CPT, chunk, 0)
    plsc.subcore_barrier()
    pltpu.sync_copy(acc_sh.at[pl.ds(lo, RPT)],
                    out_hbm.at[pl.ds(c * NPAD + lo, RPT)])


_deg_kernel = functools.partial(
    pl.kernel,
    out_type=jax.ShapeDtypeStruct((NC * NPAD,), jnp.float32),
    mesh=_sc_mesh(),
    compiler_params=_SC_PARAMS,
    scratch_types=[
        pltpu.VMEM((CPT, CH), jnp.int32),
        pltpu.VMEM((CH,), jnp.float32),
        pltpu.VMEM_SHARED((NPAD,), jnp.float32),
    ],
)(_deg_body)


def _prop_body(g_hbm, ei_hbm, zeros_hbm, out_hbm,
               src_v, dst_v, rows_v, acc_sh, *sems):
    c = lax.axis_index("c")
    s = lax.axis_index("s")
    wid = c * NS + s
    lo = s * RPT
    pltpu.async_copy(zeros_hbm.at[pl.ds(lo, RPT)], acc_sh.at[pl.ds(lo, RPT)],
                     sems[0])
    pltpu.async_copy(ei_hbm.at[0, wid], src_v, sems[1])
    pltpu.async_copy(ei_hbm.at[1, wid], dst_v, sems[2])
    pltpu.make_async_copy(zeros_hbm.at[pl.ds(lo, RPT)],
                          acc_sh.at[pl.ds(lo, RPT)], sems[0]).wait()
    pltpu.make_async_copy(ei_hbm.at[0, wid], src_v, sems[1]).wait()
    pltpu.make_async_copy(ei_hbm.at[1, wid], dst_v, sems[2]).wait()
    plsc.subcore_barrier()

    for b in range(NBUF):
        pltpu.async_copy(g_hbm.at[src_v.at[b]], rows_v.at[b], sems[b])

    def group(i, carry):
        base = i * NBUF
        for b in range(NBUF):
            cch = base + b
            pltpu.make_async_copy(g_hbm.at[src_v.at[cch]], rows_v.at[b],
                                  sems[b]).wait()
            pltpu.sync_copy(rows_v.at[b], acc_sh.at[dst_v.at[cch]], add=True)
            pltpu.async_copy(g_hbm.at[src_v.at[cch + NBUF]], rows_v.at[b],
                             sems[b])
        return carry

    lax.fori_loop(0, CPT // NBUF - 1, group, 0)
    for b in range(NBUF):
        cch = CPT - NBUF + b
        pltpu.make_async_copy(g_hbm.at[src_v.at[cch]], rows_v.at[b],
                              sems[b]).wait()
        pltpu.sync_copy(rows_v.at[b], acc_sh.at[dst_v.at[cch]], add=True)

    plsc.subcore_barrier()
    pltpu.sync_copy(acc_sh.at[pl.ds(lo, RPT)],
                    out_hbm.at[pl.ds(c * NPAD + lo, RPT)])


_prop8 = functools.partial(
    pl.kernel,
    out_type=jax.ShapeDtypeStruct((NC * NPAD, 8), jnp.float32),
    mesh=_sc_mesh(),
    compiler_params=_SC_PARAMS,
    scratch_types=[
        pltpu.VMEM((CPT, CH), jnp.int32),
        pltpu.VMEM((CPT, CH), jnp.int32),
        pltpu.VMEM((NBUF, CH, 8), jnp.float32),
        pltpu.VMEM_SHARED((NPAD, 8), jnp.float32),
    ] + [pltpu.SemaphoreType.DMA] * NBUF,
)(_prop_body)


def _elu(v):
    return jnp.where(v > 0, v, jnp.exp(v) - 1.0)


def _halves(acc_ref):
    return acc_ref[pl.ds(0, NPAD), :] + acc_ref[pl.ds(NPAD, NPAD), :]


def _full(F):
    return pl.BlockSpec((NPAD, F), lambda i: (0, 0))


def _two(F):
    return pl.BlockSpec((2 * NPAD, F), lambda i: (0, 0))


def _t1_body(dg_ref, x_ref, w_ref, dinv_ref, g_ref):
    deg = _halves(dg_ref) + 1.0
    dinv = 1.0 / jnp.sqrt(deg)
    dinv_ref[...] = dinv
    h = jnp.dot(x_ref[...], w_ref[...], preferred_element_type=jnp.float32)
    g_ref[...] = dinv * h


def _t1(dg, x, W1):
    return pl.pallas_call(
        _t1_body,
        grid=(1,),
        in_specs=[_two(1), _full(D), pl.BlockSpec((D, 8), lambda i: (0, 0))],
        out_specs=[_full(1), _full(8)],
        out_shape=[
            jax.ShapeDtypeStruct((NPAD, 1), jnp.float32),
            jax.ShapeDtypeStruct((NPAD, 8), jnp.float32),
        ],
    )(dg, x, W1)


def _tpost_body(acc_ref, g_ref, dinv_ref, b_ref, o_ref):
    dinv = dinv_ref[...]
    act = dinv * (_halves(acc_ref) + g_ref[...]) + b_ref[...]
    o_ref[...] = dinv * _elu(act)


def _tpost(acc, g, dinv, b):
    return pl.pallas_call(
        _tpost_body,
        grid=(1,),
        in_specs=[_two(8), _full(8), _full(1),
                  pl.BlockSpec((1, 8), lambda i: (0, 0))],
        out_specs=_full(8),
        out_shape=jax.ShapeDtypeStruct((NPAD, 8), jnp.float32),
    )(acc, g, dinv, b)


def _tmm2_body(acc_ref, u_ref, dinv_ref, b_ref, w2_ref, w3_ref, o_ref):
    dinv = dinv_ref[...]
    t = dinv * (_halves(acc_ref) + u_ref[...])
    h = _elu(jnp.dot(t, w2_ref[...], preferred_element_type=jnp.float32)
             + b_ref[...])
    o_ref[...] = dinv * jnp.dot(h, w3_ref[...],
                                preferred_element_type=jnp.float32)


def _tmm2(acc, u, dinv, b, W2, W3):
    return pl.pallas_call(
        _tmm2_body,
        grid=(1,),
        in_specs=[_two(8), _full(8), _full(1),
                  pl.BlockSpec((1, 16), lambda i: (0, 0)),
                  pl.BlockSpec((8, 16), lambda i: (0, 0)),
                  pl.BlockSpec((16, 8), lambda i: (0, 0))],
        out_specs=_full(8),
        out_shape=jax.ShapeDtypeStruct((NPAD, 8), jnp.float32),
    )(acc, u, dinv, b, W2, W3)


def _tfin_body(acc_ref, u_ref, dinv_ref, b_ref, w_ref, o_ref):
    t = dinv_ref[...] * (_halves(acc_ref) + u_ref[...])
    act = jnp.dot(t, w_ref[...], preferred_element_type=jnp.float32) + b_ref[...]
    m = jnp.max(act, axis=1, keepdims=True)
    sft = act - m
    o_ref[...] = sft - jnp.log(jnp.sum(jnp.exp(sft), axis=1, keepdims=True))


def _tfin(acc, u, dinv, b, W):
    return pl.pallas_call(
        _tfin_body,
        grid=(1,),
        in_specs=[_two(8), _full(8), _full(1),
                  pl.BlockSpec((1, 2), lambda i: (0, 0)),
                  pl.BlockSpec((8, 2), lambda i: (0, 0))],
        out_specs=pl.BlockSpec((NPAD, 2), lambda i: (0, 0)),
        out_shape=jax.ShapeDtypeStruct((N, 2), jnp.float32),
    )(acc, u, dinv, b, W)


_ZEROS1 = np.zeros((NPAD,), np.float32)
_ZEROS8 = np.zeros((NPAD, 8), np.float32)


def kernel(x, edge_index, laplacian_index, laplacian_weight,
           W1, b1, W2, b2, W3, b3, W4, b4):
    del laplacian_index, laplacian_weight
    er = edge_index.reshape(2, NW, CPT, CH)

    deg2 = _deg_kernel(er, _ZEROS1)
    dinv, g1 = _t1(deg2.reshape(2 * NPAD, 1), x, W1)

    acc = _prop8(g1, er, _ZEROS8)
    u2 = _tpost(acc.reshape(2 * NPAD, 8), g1, dinv, b1.reshape(1, 8))

    acc = _prop8(u2, er, _ZEROS8)
    g3 = _tmm2(acc.reshape(2 * NPAD, 8), u2, dinv, b2.reshape(1, 16), W2, W3)

    acc = _prop8(g3, er, _ZEROS8)
    u4 = _tpost(acc.reshape(2 * NPAD, 8), g3, dinv, b3.reshape(1, 8))

    acc = _prop8(u4, er, _ZEROS8)
    return _tfin(acc.reshape(2 * NPAD, 8), u4, dinv, b4.reshape(1, 2), W4)


# speedup vs baseline: 1.0626x; 1.0626x over previous
import functools

import numpy as np
import jax
import jax.numpy as jnp
from jax import lax
from jax.experimental import pallas as pl
from jax.experimental.pallas import tpu as pltpu
from jax.experimental.pallas import tpu_sc as plsc

N = 10000
E = 320000
D = 128

NC, NS, L = 2, 16, 16
NW = NC * NS
CH = 400
CPT = 25
NPAD = 10240
RPT = NPAD // NS
NBUF = 5


def _sc_mesh():
    return plsc.VectorSubcoreMesh(core_axis_name="c", subcore_axis_name="s",
                                  num_cores=NC, num_subcores=NS)


_SC_PARAMS = pltpu.CompilerParams(use_tc_tiling_on_sc=False)


def _deg_body(ei_hbm, zeros_hbm, out_hbm, dst_v, ones_v, acc_sh):
    c = lax.axis_index("c")
    s = lax.axis_index("s")
    wid = c * NS + s
    lo = s * RPT
    pltpu.sync_copy(zeros_hbm.at[pl.ds(lo, RPT)], acc_sh.at[pl.ds(lo, RPT)])
    pltpu.sync_copy(ei_hbm.at[1, wid], dst_v)
    for i in range(CH // L):
        ones_v[pl.ds(i * L, L)] = jnp.full((L,), 1.0, jnp.float32)
    plsc.subcore_barrier()

    def chunk(j, carry):
        pltpu.sync_copy(ones_v, acc_sh.at[dst_v.at[j]], add=True)
        return carry

    lax.fori_loop(0, CPT, chunk, 0)
    plsc.subcore_barrier()
    pltpu.sync_copy(acc_sh.at[pl.ds(lo, RPT)],
                    out_hbm.at[pl.ds(c * NPAD + lo, RPT)])


_deg_kernel = functools.partial(
    pl.kernel,
    out_type=jax.ShapeDtypeStruct((NC * NPAD,), jnp.float32),
    mesh=_sc_mesh(),
    compiler_params=_SC_PARAMS,
    scratch_types=[
        pltpu.VMEM((CPT, CH), jnp.int32),
        pltpu.VMEM((CH,), jnp.float32),
        pltpu.VMEM_SHARED((NPAD,), jnp.float32),
    ],
)(_deg_body)


def _prop_body(g_hbm, ei_hbm, zeros_hbm, out_hbm,
               src_v, dst_v, rows_v, acc_sh, *sems):
    c = lax.axis_index("c")
    s = lax.axis_index("s")
    wid = c * NS + s
    lo = s * RPT
    pltpu.async_copy(zeros_hbm.at[pl.ds(lo, RPT)], acc_sh.at[pl.ds(lo, RPT)],
                     sems[0])
    pltpu.async_copy(ei_hbm.at[0, wid], src_v, sems[1])
    pltpu.async_copy(ei_hbm.at[1, wid], dst_v, sems[2])
    pltpu.make_async_copy(zeros_hbm.at[pl.ds(lo, RPT)],
                          acc_sh.at[pl.ds(lo, RPT)], sems[0]).wait()
    pltpu.make_async_copy(ei_hbm.at[0, wid], src_v, sems[1]).wait()
    pltpu.make_async_copy(ei_hbm.at[1, wid], dst_v, sems[2]).wait()
    plsc.subcore_barrier()

    for b in range(NBUF):
        pltpu.async_copy(g_hbm.at[src_v.at[b]], rows_v.at[b], sems[b])

    def group(i, carry):
        base = i * NBUF
        for b in range(NBUF):
            cch = base + b
            pltpu.make_async_copy(g_hbm.at[src_v.at[cch]], rows_v.at[b],
                                  sems[b]).wait()
            pltpu.sync_copy(rows_v.at[b], acc_sh.at[dst_v.at[cch]], add=True)
            pltpu.async_copy(g_hbm.at[src_v.at[cch + NBUF]], rows_v.at[b],
                             sems[b])
        return carry

    lax.fori_loop(0, CPT // NBUF - 1, group, 0)
    for b in range(NBUF):
        cch = CPT - NBUF + b
        pltpu.make_async_copy(g_hbm.at[src_v.at[cch]], rows_v.at[b],
                              sems[b]).wait()
        pltpu.sync_copy(rows_v.at[b], acc_sh.at[dst_v.at[cch]], add=True)

    plsc.subcore_barrier()
    pltpu.sync_copy(acc_sh.at[pl.ds(lo, RPT)],
                    out_hbm.at[pl.ds(c * NPAD + lo, RPT)])


_prop8 = functools.partial(
    pl.kernel,
    out_type=jax.ShapeDtypeStruct((NC * NPAD, 8), jnp.float32),
    mesh=_sc_mesh(),
    compiler_params=_SC_PARAMS,
    scratch_types=[
        pltpu.VMEM((CPT, CH), jnp.int32),
        pltpu.VMEM((CPT, CH), jnp.int32),
        pltpu.VMEM((NBUF, CH, 8), jnp.float32),
        pltpu.VMEM_SHARED((NPAD, 8), jnp.float32),
    ] + [pltpu.SemaphoreType.DMA] * NBUF,
)(_prop_body)


def _elu(v):
    return jnp.where(v > 0, v, jnp.exp(v) - 1.0)


def _fullspec(r, c):
    return pl.BlockSpec((r, c), lambda i: (0, 0))


def _unpack_acc(acc_ref):
    a = acc_ref[...].reshape(2, NPAD, 8)
    return a[0] + a[1]


def _t1_body(dg_ref, x_ref, w_ref, dinv_ref, g_ref):
    dg = dg_ref[...]
    deg = (dg[0, :] + dg[1, :] + 1.0).reshape(NPAD, 1)
    dinv = 1.0 / jnp.sqrt(deg)
    dinv_ref[...] = dinv
    h = jnp.dot(x_ref[...], w_ref[...], preferred_element_type=jnp.float32)
    g_ref[...] = dinv * h


def _t1(dg, x, W1):
    return pl.pallas_call(
        _t1_body,
        grid=(1,),
        in_specs=[_fullspec(2, NPAD), _fullspec(NPAD, D), _fullspec(D, 8)],
        out_specs=[_fullspec(NPAD, 1), _fullspec(NPAD, 8)],
        out_shape=[
            jax.ShapeDtypeStruct((NPAD, 1), jnp.float32),
            jax.ShapeDtypeStruct((NPAD, 8), jnp.float32),
        ],
    )(dg, x, W1)


def _tpost_body(acc_ref, g_ref, dinv_ref, b_ref, o_ref):
    dinv = dinv_ref[...]
    act = dinv * (_unpack_acc(acc_ref) + g_ref[...]) + b_ref[...]
    o_ref[...] = dinv * _elu(act)


def _tpost(acc, g, dinv, b):
    return pl.pallas_call(
        _tpost_body,
        grid=(1,),
        in_specs=[_fullspec(2 * NPAD, 8), _fullspec(NPAD, 8),
                  _fullspec(NPAD, 1), _fullspec(1, 8)],
        out_specs=_fullspec(NPAD, 8),
        out_shape=jax.ShapeDtypeStruct((NPAD, 8), jnp.float32),
    )(acc, g, dinv, b)


def _tmm2_body(acc_ref, u_ref, dinv_ref, b_ref, w2_ref, w3_ref, o_ref):
    dinv = dinv_ref[...]
    t = dinv * (_unpack_acc(acc_ref) + u_ref[...])
    h = _elu(jnp.dot(t, w2_ref[...], preferred_element_type=jnp.float32)
             + b_ref[...])
    o_ref[...] = dinv * jnp.dot(h, w3_ref[...],
                                preferred_element_type=jnp.float32)


def _tmm2(acc, u, dinv, b, W2, W3):
    return pl.pallas_call(
        _tmm2_body,
        grid=(1,),
        in_specs=[_fullspec(2 * NPAD, 8), _fullspec(NPAD, 8),
                  _fullspec(NPAD, 1), _fullspec(1, 16),
                  _fullspec(8, 16), _fullspec(16, 8)],
        out_specs=_fullspec(NPAD, 8),
        out_shape=jax.ShapeDtypeStruct((NPAD, 8), jnp.float32),
    )(acc, u, dinv, b, W2, W3)


def _tfin_body(acc_ref, u_ref, dinv_ref, b_ref, w_ref, o_ref):
    t = dinv_ref[...] * (_unpack_acc(acc_ref) + u_ref[...])
    act = jnp.dot(t, w_ref[...], preferred_element_type=jnp.float32) + b_ref[...]
    m = jnp.max(act, axis=1, keepdims=True)
    sft = act - m
    o_ref[...] = sft - jnp.log(jnp.sum(jnp.exp(sft), axis=1, keepdims=True))


def _tfin(acc, u, dinv, b, W):
    return pl.pallas_call(
        _tfin_body,
        grid=(1,),
        in_specs=[_fullspec(2 * NPAD, 8), _fullspec(NPAD, 8),
                  _fullspec(NPAD, 1), _fullspec(1, 2), _fullspec(8, 2)],
        out_specs=pl.BlockSpec((NPAD, 2), lambda i: (0, 0)),
        out_shape=jax.ShapeDtypeStruct((N, 2), jnp.float32),
    )(acc, u, dinv, b, W)


_ZEROS1 = np.zeros((NPAD,), np.float32)
_ZEROS8 = np.zeros((NPAD, 8), np.float32)


def kernel(x, edge_index, laplacian_index, laplacian_weight,
           W1, b1, W2, b2, W3, b3, W4, b4):
    del laplacian_index, laplacian_weight
    er = edge_index.reshape(2, NW, CPT, CH)

    def sc(a):
        return a

    def tc(acc):
        return acc

    deg2 = _deg_kernel(er, _ZEROS1)
    dinv, g1 = _t1(deg2.reshape(2, NPAD), x, W1)

    acc = _prop8(sc(g1), er, _ZEROS8)
    u2 = _tpost(tc(acc), g1, dinv, b1.reshape(1, 8))

    acc = _prop8(sc(u2), er, _ZEROS8)
    g3 = _tmm2(tc(acc), u2, dinv, b2.reshape(1, 16), W2, W3)

    acc = _prop8(sc(g3), er, _ZEROS8)
    u4 = _tpost(tc(acc), g3, dinv, b3.reshape(1, 8))

    acc = _prop8(sc(u4), er, _ZEROS8)
    return _tfin(tc(acc), u4, dinv, b4.reshape(1, 2), W4)
